# baseline (speedup 1.0000x reference)
import functools

import jax
import jax.numpy as jnp
from jax import lax
from jax.experimental import pallas as pl
from jax.experimental.pallas import tpu as pltpu
from jax.experimental.pallas import tpu_sc as plsc

N = 10000
F = 128
E = 320000
NC = 2
NS = 16
NW = NC * NS
CHUNK = 64
NCHUNK = 160
EPT = NCHUNK * CHUNK
EPAD = NW * EPT
ROWS_PT = 632
NP = NS * ROWS_PT
LANES = 16
NBUF = 2


def _spmm_body(support, src, dst, w, zeros, out,
               src_v, dst0, dst1, w0, w1, rows0, rows1, acc,
               rsem0, rsem1, isem0, isem1):
    c = lax.axis_index("c")
    s = lax.axis_index("s")
    wid = c * NS + s
    rows = (rows0, rows1)
    dstb = (dst0, dst1)
    wb = (w0, w1)
    rsem = (rsem0, rsem1)
    isem = (isem0, isem1)
    e0 = wid * EPT

    pltpu.sync_copy(zeros, acc.at[pl.ds(s * ROWS_PT, ROWS_PT)])
    pltpu.sync_copy(src.at[pl.ds(e0, EPT)], src_v)
    plsc.subcore_barrier()

    def _start(g, b):
        csl = src_v.at[pl.ds(g * CHUNK, CHUNK)]
        pltpu.async_copy(support.at[csl], rows[b], rsem[b])
        pltpu.async_copy(dst.at[pl.ds(e0 + g * CHUNK, CHUNK)], dstb[b],
                         isem[b])
        pltpu.async_copy(w.at[pl.ds(e0 + g * CHUNK, CHUNK)], wb[b], isem[b])

    def _wait(g, b):
        csl = src_v.at[pl.ds(g * CHUNK, CHUNK)]
        pltpu.make_async_copy(support.at[csl], rows[b], rsem[b]).wait()
        pltpu.make_async_copy(dst.at[pl.ds(e0 + g * CHUNK, CHUNK)],
                              dstb[b], isem[b]).wait()
        pltpu.make_async_copy(w.at[pl.ds(e0 + g * CHUNK, CHUNK)],
                              wb[b], isem[b]).wait()

    for b in range(NBUF):
        _start(b, b)

    def g_body(g2, carry):
        for b in range(NBUF):
            g = g2 * NBUF + b
            _wait(g, b)

            def e_body(e16, carry2, _b=b):
                wv = wb[_b][pl.ds(e16 * LANES, LANES)]
                for j in range(LANES):
                    we = wv[j]
                    e = e16 * LANES + j
                    for f in range(F // LANES):
                        sl = pl.ds(f * LANES, LANES)
                        rows[_b][e, sl] = rows[_b][e, sl] * we
                return carry2

            lax.fori_loop(0, CHUNK // LANES, e_body, 0)
            pltpu.sync_copy(rows[b], acc.at[dstb[b]], add=True)

            @pl.when(g2 < NCHUNK // NBUF - 1)
            def _(b=b, g=g):
                _start(g + NBUF, b)
        return carry

    lax.fori_loop(0, NCHUNK // NBUF, g_body, 0)
    plsc.subcore_barrier()
    pltpu.sync_copy(acc.at[pl.ds(s * ROWS_PT, ROWS_PT)],
                    out.at[c, pl.ds(s * ROWS_PT, ROWS_PT)])


_spmm = functools.partial(
    pl.kernel,
    out_type=jax.ShapeDtypeStruct((NC, NP, F), jnp.float32),
    mesh=plsc.VectorSubcoreMesh(core_axis_name="c", subcore_axis_name="s"),
    scratch_types=[
        pltpu.VMEM((EPT,), jnp.int32),
        pltpu.VMEM((CHUNK,), jnp.int32),
        pltpu.VMEM((CHUNK,), jnp.int32),
        pltpu.VMEM((CHUNK,), jnp.float32),
        pltpu.VMEM((CHUNK,), jnp.float32),
        pltpu.VMEM((CHUNK, F), jnp.float32),
        pltpu.VMEM((CHUNK, F), jnp.float32),
        pltpu.VMEM_SHARED((NP, F), jnp.float32),
        pltpu.SemaphoreType.DMA,
        pltpu.SemaphoreType.DMA,
        pltpu.SemaphoreType.DMA,
        pltpu.SemaphoreType.DMA,
    ],
)(_spmm_body)


BM = 1000


def _mm_body(x_ref, w_ref, o_ref):
    o_ref[...] = jnp.dot(x_ref[...], w_ref[...],
                         preferred_element_type=jnp.float32)


def _matmul(x, W):
    return pl.pallas_call(
        _mm_body,
        grid=(N // BM,),
        in_specs=[pl.BlockSpec((BM, F), lambda i: (i, 0)),
                  pl.BlockSpec((F, F), lambda i: (0, 0))],
        out_specs=pl.BlockSpec((BM, F), lambda i: (i, 0)),
        out_shape=jax.ShapeDtypeStruct((N, F), jnp.float32),
    )(x, W)


def _prologue(p_ref, b_ref):
    return jnp.maximum(p_ref[0] + p_ref[1] + b_ref[...], 0.0)


def _supp_body(p_ref, b_ref, w_ref, s_ref):
    s_ref[...] = jnp.dot(_prologue(p_ref, b_ref), w_ref[...],
                         preferred_element_type=jnp.float32)


def _fuse_body(p_ref, b_ref, w_ref, h_ref, s_ref):
    h = _prologue(p_ref, b_ref)
    h_ref[...] = h
    s_ref[...] = jnp.dot(h, w_ref[...], preferred_element_type=jnp.float32)


def _final_body(p_ref, b_ref, h_ref):
    h_ref[...] = _prologue(p_ref, b_ref)


_P_SPEC = pl.BlockSpec((NC, BM, F), lambda i: (0, i, 0))
_B_SPEC = pl.BlockSpec((1, F), lambda i: (0, 0))
_W_SPEC = pl.BlockSpec((F, F), lambda i: (0, 0))
_H_SPEC = pl.BlockSpec((BM, F), lambda i: (i, 0))
_HS = jax.ShapeDtypeStruct((N, F), jnp.float32)


def _support_only(p, b, W):
    return pl.pallas_call(
        _supp_body, grid=(N // BM,),
        in_specs=[_P_SPEC, _B_SPEC, _W_SPEC],
        out_specs=_H_SPEC, out_shape=_HS,
    )(p, b, W)


def _fuse(p, b, W):
    return pl.pallas_call(
        _fuse_body, grid=(N // BM,),
        in_specs=[_P_SPEC, _B_SPEC, _W_SPEC],
        out_specs=(_H_SPEC, _H_SPEC), out_shape=(_HS, _HS),
    )(p, b, W)


def _final(p, b):
    return pl.pallas_call(
        _final_body, grid=(N // BM,),
        in_specs=[_P_SPEC, _B_SPEC],
        out_specs=_H_SPEC, out_shape=_HS,
    )(p, b)


def kernel(x, edge_index, edge_weight, W1, b1, W2, b2, W3, b3):
    pad_i = jnp.zeros((EPAD - E,), jnp.int32)
    src = jnp.concatenate([edge_index[0], pad_i])
    dst = jnp.concatenate([edge_index[1], pad_i])
    w2d = jnp.concatenate(
        [edge_weight, jnp.zeros((EPAD - E,), jnp.float32)])
    zeros = jnp.zeros((ROWS_PT, F), jnp.float32)
    b1r, b2r, b3r = (b.reshape(1, F) for b in (b1, b2, b3))

    t = _matmul(x, W1)
    p = _spmm(t, src, dst, w2d, zeros)
    t = _support_only(p, b1r, W2)
    p = _spmm(t, src, dst, w2d, zeros)
    h2, t = _fuse(p, b2r, W3)
    p = _spmm(t, src, dst, w2d, zeros)
    h3, t = _fuse(p, b3r, W3)
    p = _spmm(t, src, dst, w2d, zeros)
    h4, t = _fuse(p, b3r, W3)
    p = _spmm(t, src, dst, w2d, zeros)
    h5 = _final(p, b3r)
    return (h2, h3, h4, h5)


# speedup vs baseline: 3.1166x; 3.1166x over previous
import functools

import jax
import jax.numpy as jnp
from jax import lax
from jax.experimental import pallas as pl
from jax.experimental.pallas import tpu as pltpu
from jax.experimental.pallas import tpu_sc as plsc

N = 10000
F = 128
E = 320000
NC = 2
NS = 16
NW = NC * NS
CHUNK = 80
NCHUNK = 125
EPT = NCHUNK * CHUNK
EPAD = NW * EPT
ROWS_PT = 632
NP = NS * ROWS_PT
LANES = 16
NBUF = 2


def _spmm_body(support, src, dst, w, zeros, out,
               src_v, w_v, dst0, dst1, rows0, rows1, acc,
               rsem0, rsem1, isem0, isem1):
    c = lax.axis_index("c")
    s = lax.axis_index("s")
    wid = c * NS + s
    rows = (rows0, rows1)
    dstb = (dst0, dst1)
    rsem = (rsem0, rsem1)
    isem = (isem0, isem1)
    e0 = wid * EPT

    pltpu.sync_copy(zeros, acc.at[pl.ds(s * ROWS_PT, ROWS_PT)])
    pltpu.sync_copy(src.at[pl.ds(e0, EPT)], src_v)
    pltpu.sync_copy(w.at[pl.ds(e0, EPT)], w_v)
    plsc.subcore_barrier()

    def _start(g, b):
        csl = src_v.at[pl.ds(g * CHUNK, CHUNK)]
        pltpu.async_copy(support.at[csl], rows[b], rsem[b])
        pltpu.async_copy(dst.at[pl.ds(e0 + g * CHUNK, CHUNK)], dstb[b],
                         isem[b])

    def _wait(b):
        pltpu.make_async_copy(support.at[pl.ds(0, CHUNK)],
                              rows[b], rsem[b]).wait()
        pltpu.make_async_copy(dst.at[pl.ds(0, CHUNK)], dstb[b],
                              isem[b]).wait()

    def _process(g, b):
        def e_body(e16, carry2, _b=b, _g=g):
            wv = w_v[pl.ds(_g * CHUNK + e16 * LANES, LANES)]
            for j in range(LANES):
                we = wv[j]
                e = e16 * LANES + j
                for f in range(F // LANES):
                    sl = pl.ds(f * LANES, LANES)
                    rows[_b][e, sl] = rows[_b][e, sl] * we
            return carry2

        lax.fori_loop(0, CHUNK // LANES, e_body, 0)
        pltpu.sync_copy(rows[b], acc.at[dstb[b]], add=True)

    for b in range(NBUF):
        _start(b, b)

    def g_body(g2, carry):
        for b in range(NBUF):
            g = g2 * NBUF + b
            _wait(b)
            _process(g, b)
            if b == 0:
                _start(g + NBUF, b)
            else:
                @pl.when(g2 < (NCHUNK - 1) // NBUF - 1)
                def _(b=b, g=g):
                    _start(g + NBUF, b)
        return carry

    lax.fori_loop(0, (NCHUNK - 1) // NBUF, g_body, 0)
    _wait(0)
    _process(NCHUNK - 1, 0)

    plsc.subcore_barrier()
    pltpu.sync_copy(acc.at[pl.ds(s * ROWS_PT, ROWS_PT)],
                    out.at[c, pl.ds(s * ROWS_PT, ROWS_PT)])


_spmm = functools.partial(
    pl.kernel,
    out_type=jax.ShapeDtypeStruct((NC, NP, F), jnp.float32),
    mesh=plsc.VectorSubcoreMesh(core_axis_name="c", subcore_axis_name="s"),
    scratch_types=[
        pltpu.VMEM((EPT,), jnp.int32),
        pltpu.VMEM((EPT,), jnp.float32),
        pltpu.VMEM((CHUNK,), jnp.int32),
        pltpu.VMEM((CHUNK,), jnp.int32),
        pltpu.VMEM((CHUNK, F), jnp.float32),
        pltpu.VMEM((CHUNK, F), jnp.float32),
        pltpu.VMEM_SHARED((NP, F), jnp.float32),
        pltpu.SemaphoreType.DMA,
        pltpu.SemaphoreType.DMA,
        pltpu.SemaphoreType.DMA,
        pltpu.SemaphoreType.DMA,
    ],
)(_spmm_body)


BM = 1000


def _mm_body(x_ref, w_ref, o_ref):
    o_ref[...] = jnp.dot(x_ref[...], w_ref[...],
                         preferred_element_type=jnp.float32)


def _matmul(x, W):
    return pl.pallas_call(
        _mm_body,
        grid=(N // BM,),
        in_specs=[pl.BlockSpec((BM, F), lambda i: (i, 0)),
                  pl.BlockSpec((F, F), lambda i: (0, 0))],
        out_specs=pl.BlockSpec((BM, F), lambda i: (i, 0)),
        out_shape=jax.ShapeDtypeStruct((N, F), jnp.float32),
    )(x, W)


def _prologue(p_ref, b_ref):
    return jnp.maximum(p_ref[0] + p_ref[1] + b_ref[...], 0.0)


def _supp_body(p_ref, b_ref, w_ref, s_ref):
    s_ref[...] = jnp.dot(_prologue(p_ref, b_ref), w_ref[...],
                         preferred_element_type=jnp.float32)


def _fuse_body(p_ref, b_ref, w_ref, h_ref, s_ref):
    h = _prologue(p_ref, b_ref)
    h_ref[...] = h
    s_ref[...] = jnp.dot(h, w_ref[...], preferred_element_type=jnp.float32)


def _final_body(p_ref, b_ref, h_ref):
    h_ref[...] = _prologue(p_ref, b_ref)


_P_SPEC = pl.BlockSpec((NC, BM, F), lambda i: (0, i, 0))
_B_SPEC = pl.BlockSpec((1, F), lambda i: (0, 0))
_W_SPEC = pl.BlockSpec((F, F), lambda i: (0, 0))
_H_SPEC = pl.BlockSpec((BM, F), lambda i: (i, 0))
_HS = jax.ShapeDtypeStruct((N, F), jnp.float32)


def _support_only(p, b, W):
    return pl.pallas_call(
        _supp_body, grid=(N // BM,),
        in_specs=[_P_SPEC, _B_SPEC, _W_SPEC],
        out_specs=_H_SPEC, out_shape=_HS,
    )(p, b, W)


def _fuse(p, b, W):
    return pl.pallas_call(
        _fuse_body, grid=(N // BM,),
        in_specs=[_P_SPEC, _B_SPEC, _W_SPEC],
        out_specs=(_H_SPEC, _H_SPEC), out_shape=(_HS, _HS),
    )(p, b, W)


def _final(p, b):
    return pl.pallas_call(
        _final_body, grid=(N // BM,),
        in_specs=[_P_SPEC, _B_SPEC],
        out_specs=_H_SPEC, out_shape=_HS,
    )(p, b)


def kernel(x, edge_index, edge_weight, W1, b1, W2, b2, W3, b3):
    src = edge_index[0]
    dst = edge_index[1]
    w2d = edge_weight
    zeros = jnp.zeros((ROWS_PT, F), jnp.float32)
    b1r, b2r, b3r = (b.reshape(1, F) for b in (b1, b2, b3))

    t = _matmul(x, W1)
    p = _spmm(t, src, dst, w2d, zeros)
    t = _support_only(p, b1r, W2)
    p = _spmm(t, src, dst, w2d, zeros)
    h2, t = _fuse(p, b2r, W3)
    p = _spmm(t, src, dst, w2d, zeros)
    h3, t = _fuse(p, b3r, W3)
    p = _spmm(t, src, dst, w2d, zeros)
    h4, t = _fuse(p, b3r, W3)
    p = _spmm(t, src, dst, w2d, zeros)
    h5 = _final(p, b3r)
    return (h2, h3, h4, h5)


# speedup vs baseline: 3.5343x; 1.1340x over previous
import functools

import jax
import jax.numpy as jnp
from jax import lax
from jax.experimental import pallas as pl
from jax.experimental.pallas import tpu as pltpu
from jax.experimental.pallas import tpu_sc as plsc

N = 10000
F = 128
E = 320000
NC = 2
NS = 16
NW = NC * NS
CHUNK = 80
NCHUNK = 125
EPT = NCHUNK * CHUNK
EPAD = NW * EPT
ROWS_PT = 632
NP = NS * ROWS_PT
LANES = 16
NBUF = 2


def _spmm_body(support, src, dst, w, zeros, out,
               src_v, dst0, dst1, dst2, w0, w1, w2, rows0, rows1, rows2, acc,
               rsem0, rsem1, rsem2, ssem0, ssem1, ssem2):
    c = lax.axis_index("c")
    s = lax.axis_index("s")
    wid = c * NS + s
    rows = (rows0, rows1, rows2)
    dstb = (dst0, dst1, dst2)
    wb = (w0, w1, w2)
    rsem = (rsem0, rsem1, rsem2)
    ssem = (ssem0, ssem1, ssem2)
    e0 = wid * EPT

    pltpu.sync_copy(zeros, acc.at[pl.ds(s * ROWS_PT, ROWS_PT)])
    pltpu.sync_copy(src.at[pl.ds(e0, EPT)], src_v)
    plsc.subcore_barrier()

    def _start(g, b):
        csl = src_v.at[pl.ds(g * CHUNK, CHUNK)]
        pltpu.async_copy(support.at[csl], rows[b], rsem[b])
        pltpu.async_copy(dst.at[pl.ds(e0 + g * CHUNK, CHUNK)], dstb[b],
                         rsem[b])
        pltpu.async_copy(w.at[pl.ds(e0 + g * CHUNK, CHUNK)], wb[b], rsem[b])

    def _drain_in(b):
        pltpu.make_async_copy(support.at[pl.ds(0, CHUNK)],
                              rows[b], rsem[b]).wait()
        pltpu.make_async_copy(dst.at[pl.ds(0, CHUNK)], dstb[b],
                              rsem[b]).wait()
        pltpu.make_async_copy(w.at[pl.ds(0, CHUNK)], wb[b], rsem[b]).wait()

    def _drain_scatter(b):
        pltpu.make_async_copy(rows[b], acc.at[pl.ds(0, CHUNK)],
                              ssem[b]).wait()

    def _scale(g, b):
        def e_body(e16, carry2, _b=b):
            wv = wb[_b][pl.ds(e16 * LANES, LANES)]
            for j in range(LANES):
                we = wv[j]
                e = e16 * LANES + j
                for f in range(F // LANES):
                    sl = pl.ds(f * LANES, LANES)
                    rows[_b][e, sl] = rows[_b][e, sl] * we
            return carry2

        lax.fori_loop(0, CHUNK // LANES, e_body, 0)

    _start(0, 0)
    _start(1, 1)

    NLOOP = (NCHUNK - 2) // 3

    def g_body(g3, carry):
        for b in range(3):
            g = g3 * 3 + b
            _drain_in(b)
            _scale(g, b)
            pltpu.async_copy(rows[b], acc.at[dstb[b]], ssem[b], add=True)
            b2 = (b + 2) % 3
            if b == 0:
                @pl.when(g3 > 0)
                def _(b2=b2):
                    _drain_scatter(b2)
            else:
                _drain_scatter(b2)
            _start(g + 2, b2)
        return carry

    lax.fori_loop(0, NLOOP, g_body, 0)

    for g, b in ((NCHUNK - 2, 0), (NCHUNK - 1, 1)):
        _drain_in(b)
        _scale(g, b)
        pltpu.async_copy(rows[b], acc.at[dstb[b]], ssem[b], add=True)
    _drain_scatter(2)
    _drain_scatter(0)
    _drain_scatter(1)

    plsc.subcore_barrier()
    pltpu.sync_copy(acc.at[pl.ds(s * ROWS_PT, ROWS_PT)],
                    out.at[c, pl.ds(s * ROWS_PT, ROWS_PT)])


_spmm = functools.partial(
    pl.kernel,
    out_type=jax.ShapeDtypeStruct((NC, NP, F), jnp.float32),
    mesh=plsc.VectorSubcoreMesh(core_axis_name="c", subcore_axis_name="s"),
    scratch_types=[
        pltpu.VMEM((EPT,), jnp.int32),
        pltpu.VMEM((CHUNK,), jnp.int32),
        pltpu.VMEM((CHUNK,), jnp.int32),
        pltpu.VMEM((CHUNK,), jnp.int32),
        pltpu.VMEM((CHUNK,), jnp.float32),
        pltpu.VMEM((CHUNK,), jnp.float32),
        pltpu.VMEM((CHUNK,), jnp.float32),
        pltpu.VMEM((CHUNK, F), jnp.float32),
        pltpu.VMEM((CHUNK, F), jnp.float32),
        pltpu.VMEM((CHUNK, F), jnp.float32),
        pltpu.VMEM_SHARED((NP, F), jnp.float32),
        pltpu.SemaphoreType.DMA,
        pltpu.SemaphoreType.DMA,
        pltpu.SemaphoreType.DMA,
        pltpu.SemaphoreType.DMA,
        pltpu.SemaphoreType.DMA,
        pltpu.SemaphoreType.DMA,
    ],
)(_spmm_body)


BM = 1000


def _mm_body(x_ref, w_ref, o_ref):
    o_ref[...] = jnp.dot(x_ref[...], w_ref[...],
                         preferred_element_type=jnp.float32)


def _matmul(x, W):
    return pl.pallas_call(
        _mm_body,
        grid=(N // BM,),
        in_specs=[pl.BlockSpec((BM, F), lambda i: (i, 0)),
                  pl.BlockSpec((F, F), lambda i: (0, 0))],
        out_specs=pl.BlockSpec((BM, F), lambda i: (i, 0)),
        out_shape=jax.ShapeDtypeStruct((N, F), jnp.float32),
    )(x, W)


def _prologue(p_ref, b_ref):
    return jnp.maximum(p_ref[0] + p_ref[1] + b_ref[...], 0.0)


def _supp_body(p_ref, b_ref, w_ref, s_ref):
    s_ref[...] = jnp.dot(_prologue(p_ref, b_ref), w_ref[...],
                         preferred_element_type=jnp.float32)


def _fuse_body(p_ref, b_ref, w_ref, h_ref, s_ref):
    h = _prologue(p_ref, b_ref)
    h_ref[...] = h
    s_ref[...] = jnp.dot(h, w_ref[...], preferred_element_type=jnp.float32)


def _final_body(p_ref, b_ref, h_ref):
    h_ref[...] = _prologue(p_ref, b_ref)


_P_SPEC = pl.BlockSpec((NC, BM, F), lambda i: (0, i, 0))
_B_SPEC = pl.BlockSpec((1, F), lambda i: (0, 0))
_W_SPEC = pl.BlockSpec((F, F), lambda i: (0, 0))
_H_SPEC = pl.BlockSpec((BM, F), lambda i: (i, 0))
_HS = jax.ShapeDtypeStruct((N, F), jnp.float32)


def _support_only(p, b, W):
    return pl.pallas_call(
        _supp_body, grid=(N // BM,),
        in_specs=[_P_SPEC, _B_SPEC, _W_SPEC],
        out_specs=_H_SPEC, out_shape=_HS,
    )(p, b, W)


def _fuse(p, b, W):
    return pl.pallas_call(
        _fuse_body, grid=(N // BM,),
        in_specs=[_P_SPEC, _B_SPEC, _W_SPEC],
        out_specs=(_H_SPEC, _H_SPEC), out_shape=(_HS, _HS),
    )(p, b, W)


def _final(p, b):
    return pl.pallas_call(
        _final_body, grid=(N // BM,),
        in_specs=[_P_SPEC, _B_SPEC],
        out_specs=_H_SPEC, out_shape=_HS,
    )(p, b)


def kernel(x, edge_index, edge_weight, W1, b1, W2, b2, W3, b3):
    src = edge_index[0]
    dst = edge_index[1]
    w2d = edge_weight
    zeros = jnp.zeros((ROWS_PT, F), jnp.float32)
    b1r, b2r, b3r = (b.reshape(1, F) for b in (b1, b2, b3))

    t = _matmul(x, W1)
    p = _spmm(t, src, dst, w2d, zeros)
    t = _support_only(p, b1r, W2)
    p = _spmm(t, src, dst, w2d, zeros)
    h2, t = _fuse(p, b2r, W3)
    p = _spmm(t, src, dst, w2d, zeros)
    h3, t = _fuse(p, b3r, W3)
    p = _spmm(t, src, dst, w2d, zeros)
    h4, t = _fuse(p, b3r, W3)
    p = _spmm(t, src, dst, w2d, zeros)
    h5 = _final(p, b3r)
    return (h2, h3, h4, h5)
